# Initial kernel scaffold; baseline (speedup 1.0000x reference)
import jax
import jax.numpy as jnp
from jax.experimental import pallas as pl


def kernel(input_features, expert_centroids, norm_w, norm_b, ff1_w, ff1_b, ff2_w, ff2_b):
    raise NotImplementedError("write your pallas kernel here")


# speedup vs baseline: 2.8062x; 2.8062x over previous
import functools

import jax
import jax.numpy as jnp
from jax import lax
from jax.experimental import pallas as pl
from jax.experimental.pallas import tpu as pltpu
from jax.experimental.pallas import tpu_sc as plsc

E = 16
D = 1024
DFF = 4096
N = 4096
T = 256
G = 32
NW = 32
CH = 64


def _routing_body(x_ref, c_ref, eid_ref, alpha_ref):
    aff = lax.dot_general(x_ref[...], c_ref[...],
                          (((1,), (1,)), ((), ())),
                          preferred_element_type=jnp.float32)
    eid_ref[...] = jnp.argmax(aff, axis=1).astype(jnp.int32)
    alpha_ref[...] = jax.nn.sigmoid(jnp.max(aff, axis=1))


def _routing(feats, cents):
    return pl.pallas_call(
        _routing_body,
        out_shape=(jax.ShapeDtypeStruct((N,), jnp.int32),
                   jax.ShapeDtypeStruct((N,), jnp.float32)),
    )(feats, cents)


def _sc_gather(table, idx3, k):
    d = table.shape[1]
    mesh = plsc.VectorSubcoreMesh(core_axis_name="c", subcore_axis_name="s")
    info = plsc.get_sparse_core_info()
    nc = info.num_cores

    @functools.partial(
        pl.kernel, mesh=mesh,
        out_type=jax.ShapeDtypeStruct((NW * k * CH, d), jnp.float32),
        scratch_types=[
            pltpu.VMEM((k, CH), jnp.int32),
            pltpu.VMEM((CH, d), jnp.float32),
            pltpu.SemaphoreType.DMA,
        ],
    )
    def run(table_hbm, idx_hbm, out_hbm, idx_v, rows_v, sem):
        wid = lax.axis_index("s") * nc + lax.axis_index("c")
        pltpu.sync_copy(idx_hbm.at[wid], idx_v)
        for c in range(k):
            pltpu.async_copy(table_hbm.at[idx_v.at[c]], rows_v, sem).wait()
            pltpu.sync_copy(rows_v, out_hbm.at[pl.ds(wid * k * CH + c * CH, CH)])

    return run(table, idx3)


def _ffn_body(te_ref, act_ref, x_ref, a_ref, nw_ref, nb_ref,
              w1_ref, b1_ref, w2_ref, b2_ref, out_ref):
    g = pl.program_id(0)

    @pl.when(g < act_ref[0])
    def _():
        x = x_ref[...]
        mu = jnp.mean(x, axis=1, keepdims=True)
        var = jnp.mean((x - mu) ** 2, axis=1, keepdims=True)
        xh = (x - mu) * lax.rsqrt(var + 1e-5) * nw_ref[0] + nb_ref[0]
        h = lax.dot_general(xh.astype(jnp.bfloat16), w1_ref[0],
                            (((1,), (1,)), ((), ())),
                            preferred_element_type=jnp.float32)
        h = jnp.maximum(h + b1_ref[0], 0.0)
        y = lax.dot_general(h.astype(jnp.bfloat16), w2_ref[0],
                            (((1,), (1,)), ((), ())),
                            preferred_element_type=jnp.float32)
        out_ref[...] = x + a_ref[...] * (y + b2_ref[0])


def _grouped_ffn(te, act, x_p, alpha_p, norm_w, norm_b, ff1_w, ff1_b, ff2_w, ff2_b):
    grid_spec = pltpu.PrefetchScalarGridSpec(
        num_scalar_prefetch=2,
        grid=(G,),
        in_specs=[
            pl.BlockSpec((T, D), lambda g, te, act: (g, 0)),
            pl.BlockSpec((T, 1), lambda g, te, act: (g, 0)),
            pl.BlockSpec((1, 1, D), lambda g, te, act: (te[g], 0, 0)),
            pl.BlockSpec((1, 1, D), lambda g, te, act: (te[g], 0, 0)),
            pl.BlockSpec((1, DFF, D), lambda g, te, act: (te[g], 0, 0)),
            pl.BlockSpec((1, 1, DFF), lambda g, te, act: (te[g], 0, 0)),
            pl.BlockSpec((1, D, DFF), lambda g, te, act: (te[g], 0, 0)),
            pl.BlockSpec((1, 1, D), lambda g, te, act: (te[g], 0, 0)),
        ],
        out_specs=pl.BlockSpec((T, D), lambda g, te, act: (g, 0)),
    )
    return pl.pallas_call(
        _ffn_body,
        grid_spec=grid_spec,
        out_shape=jax.ShapeDtypeStruct((G * T, D), jnp.float32),
        compiler_params=pltpu.CompilerParams(
            dimension_semantics=("arbitrary",)),
    )(te, act, x_p, alpha_p, norm_w, norm_b, ff1_w, ff1_b, ff2_w, ff2_b)


def kernel(input_features, expert_centroids, norm_w, norm_b,
           ff1_w, ff1_b, ff2_w, ff2_b):
    feats = input_features.reshape(N, D)

    eid, alpha = _routing(feats, expert_centroids)

    oh = (eid[:, None] == jnp.arange(E, dtype=jnp.int32)[None, :]).astype(jnp.int32)
    csum = jnp.cumsum(oh, axis=0)
    rank = jnp.sum((csum - oh) * oh, axis=1)
    counts = csum[-1]
    tiles_e = (counts + T - 1) // T
    tcum = jnp.cumsum(tiles_e)
    tile_start = tcum - tiles_e
    pos = tile_start[eid] * T + rank
    src_idx = jnp.zeros(G * T, jnp.int32).at[pos].set(
        jnp.arange(N, dtype=jnp.int32))
    alpha_p = jnp.zeros(G * T, jnp.float32).at[pos].set(alpha)
    n_act = tcum[-1]
    gidx = jnp.arange(G, dtype=jnp.int32)
    te_raw = jnp.searchsorted(tcum, gidx, side="right").astype(jnp.int32)
    last_e = jnp.max(jnp.where(counts > 0, jnp.arange(E, dtype=jnp.int32), 0))
    te = jnp.where(gidx < n_act, jnp.minimum(te_raw, E - 1), last_e)
    act = n_act.reshape(1).astype(jnp.int32)

    x_p = _sc_gather(feats, src_idx.reshape(NW, (G * T) // (NW * CH), CH),
                     (G * T) // (NW * CH))
    y_p = _grouped_ffn(te, act, x_p, alpha_p.reshape(G * T, 1),
                       norm_w.reshape(E, 1, D), norm_b.reshape(E, 1, D),
                       ff1_w.astype(jnp.bfloat16), ff1_b.reshape(E, 1, DFF),
                       ff2_w.astype(jnp.bfloat16), ff2_b.reshape(E, 1, D))
    out = _sc_gather(y_p, pos.reshape(NW, N // (NW * CH), CH),
                     N // (NW * CH))
    return out.reshape(input_features.shape)


# speedup vs baseline: 2.8102x; 1.0014x over previous
import functools

import jax
import jax.numpy as jnp
from jax import lax
from jax.experimental import pallas as pl
from jax.experimental.pallas import tpu as pltpu
from jax.experimental.pallas import tpu_sc as plsc

E = 16
D = 1024
DFF = 4096
N = 4096
T = 256
G = 32
NW = 32
CH = 32


def _routing_body(x_ref, c_ref, eid_ref, alpha_ref):
    aff = lax.dot_general(x_ref[...], c_ref[...],
                          (((1,), (1,)), ((), ())),
                          preferred_element_type=jnp.float32)
    eid_ref[...] = jnp.argmax(aff, axis=1).astype(jnp.int32)
    alpha_ref[...] = jax.nn.sigmoid(jnp.max(aff, axis=1))


def _routing(feats, cents):
    return pl.pallas_call(
        _routing_body,
        out_shape=(jax.ShapeDtypeStruct((N,), jnp.int32),
                   jax.ShapeDtypeStruct((N,), jnp.float32)),
    )(feats, cents)


def _sc_gather(table, idx3, k):
    d = table.shape[1]
    mesh = plsc.VectorSubcoreMesh(core_axis_name="c", subcore_axis_name="s")
    info = plsc.get_sparse_core_info()
    nc = info.num_cores

    @functools.partial(
        pl.kernel, mesh=mesh,
        out_type=jax.ShapeDtypeStruct((NW * k * CH, d), jnp.float32),
        scratch_types=[
            pltpu.VMEM((k, CH), jnp.int32),
            pltpu.VMEM((CH, d), jnp.float32),
            pltpu.VMEM((CH, d), jnp.float32),
            pltpu.SemaphoreType.DMA,
            pltpu.SemaphoreType.DMA,
        ],
    )
    def run(table_hbm, idx_hbm, out_hbm, idx_v, r0, r1, gsem, ssem):
        wid = lax.axis_index("s") * nc + lax.axis_index("c")
        base = wid * k * CH
        pltpu.sync_copy(idx_hbm.at[wid], idx_v)
        bufs = (r0, r1)
        g = [None] * k
        s = [None] * k
        g[0] = pltpu.async_copy(table_hbm.at[idx_v.at[0]], bufs[0], gsem)
        for c in range(k):
            if c + 1 < k:
                if c >= 1:
                    s[c - 1].wait()
                g[c + 1] = pltpu.async_copy(
                    table_hbm.at[idx_v.at[c + 1]], bufs[(c + 1) % 2], gsem)
            g[c].wait()
            s[c] = pltpu.async_copy(
                bufs[c % 2], out_hbm.at[pl.ds(base + c * CH, CH)], ssem)
        s[k - 1].wait()
        if k >= 2:
            s[k - 2].wait()

    return run(table, idx3)


def _ffn_body(te_ref, act_ref, x_ref, a_ref, nw_ref, nb_ref,
              w1_ref, b1_ref, w2_ref, b2_ref, out_ref):
    g = pl.program_id(0)

    @pl.when(g < act_ref[0])
    def _():
        x = x_ref[...]
        mu = jnp.mean(x, axis=1, keepdims=True)
        var = jnp.mean((x - mu) ** 2, axis=1, keepdims=True)
        xh = (x - mu) * lax.rsqrt(var + 1e-5) * nw_ref[0] + nb_ref[0]
        h = lax.dot_general(xh.astype(jnp.bfloat16), w1_ref[0],
                            (((1,), (1,)), ((), ())),
                            preferred_element_type=jnp.float32)
        h = jnp.maximum(h + b1_ref[0], 0.0)
        y = lax.dot_general(h.astype(jnp.bfloat16), w2_ref[0],
                            (((1,), (1,)), ((), ())),
                            preferred_element_type=jnp.float32)
        out_ref[...] = x + a_ref[...] * (y + b2_ref[0])


def _grouped_ffn(te, act, x_p, alpha_p, norm_w, norm_b, ff1_w, ff1_b, ff2_w, ff2_b):
    grid_spec = pltpu.PrefetchScalarGridSpec(
        num_scalar_prefetch=2,
        grid=(G,),
        in_specs=[
            pl.BlockSpec((T, D), lambda g, te, act: (g, 0)),
            pl.BlockSpec((T, 1), lambda g, te, act: (g, 0)),
            pl.BlockSpec((1, 1, D), lambda g, te, act: (te[g], 0, 0)),
            pl.BlockSpec((1, 1, D), lambda g, te, act: (te[g], 0, 0)),
            pl.BlockSpec((1, DFF, D), lambda g, te, act: (te[g], 0, 0)),
            pl.BlockSpec((1, 1, DFF), lambda g, te, act: (te[g], 0, 0)),
            pl.BlockSpec((1, D, DFF), lambda g, te, act: (te[g], 0, 0)),
            pl.BlockSpec((1, 1, D), lambda g, te, act: (te[g], 0, 0)),
        ],
        out_specs=pl.BlockSpec((T, D), lambda g, te, act: (g, 0)),
    )
    return pl.pallas_call(
        _ffn_body,
        grid_spec=grid_spec,
        out_shape=jax.ShapeDtypeStruct((G * T, D), jnp.float32),
        compiler_params=pltpu.CompilerParams(
            dimension_semantics=("arbitrary",)),
    )(te, act, x_p, alpha_p, norm_w, norm_b, ff1_w, ff1_b, ff2_w, ff2_b)


def kernel(input_features, expert_centroids, norm_w, norm_b,
           ff1_w, ff1_b, ff2_w, ff2_b):
    feats = input_features.reshape(N, D)

    eid, alpha = _routing(feats, expert_centroids)

    oh = (eid[:, None] == jnp.arange(E, dtype=jnp.int32)[None, :]).astype(jnp.int32)
    csum = jnp.cumsum(oh, axis=0)
    rank = jnp.sum((csum - oh) * oh, axis=1)
    counts = csum[-1]
    tiles_e = (counts + T - 1) // T
    tcum = jnp.cumsum(tiles_e)
    tile_start = tcum - tiles_e
    pos = tile_start[eid] * T + rank
    src_idx = jnp.zeros(G * T, jnp.int32).at[pos].set(
        jnp.arange(N, dtype=jnp.int32))
    alpha_p = jnp.zeros(G * T, jnp.float32).at[pos].set(alpha)
    n_act = tcum[-1]
    gidx = jnp.arange(G, dtype=jnp.int32)
    te_raw = jnp.searchsorted(tcum, gidx, side="right").astype(jnp.int32)
    last_e = jnp.max(jnp.where(counts > 0, jnp.arange(E, dtype=jnp.int32), 0))
    te = jnp.where(gidx < n_act, jnp.minimum(te_raw, E - 1), last_e)
    act = n_act.reshape(1).astype(jnp.int32)

    x_p = _sc_gather(feats, src_idx.reshape(NW, (G * T) // (NW * CH), CH),
                     (G * T) // (NW * CH))
    y_p = _grouped_ffn(te, act, x_p, alpha_p.reshape(G * T, 1),
                       norm_w.reshape(E, 1, D), norm_b.reshape(E, 1, D),
                       ff1_w.astype(jnp.bfloat16), ff1_b.reshape(E, 1, DFF),
                       ff2_w.astype(jnp.bfloat16), ff2_b.reshape(E, 1, D))
    out = _sc_gather(y_p, pos.reshape(NW, N // (NW * CH), CH),
                     N // (NW * CH))
    return out.reshape(input_features.shape)


# speedup vs baseline: 3.8914x; 1.3847x over previous
import functools

import jax
import jax.numpy as jnp
from jax import lax
from jax.experimental import pallas as pl
from jax.experimental.pallas import tpu as pltpu
from jax.experimental.pallas import tpu_sc as plsc

E = 16
D = 1024
DFF = 4096
N = 4096
T = 256
G = 32
NW = 32
CH = 32


def _routing_body(x_ref, c_ref, eid_ref, alpha_ref):
    aff = lax.dot_general(x_ref[...], c_ref[...],
                          (((1,), (1,)), ((), ())),
                          preferred_element_type=jnp.float32)
    eid_ref[...] = jnp.argmax(aff, axis=1).astype(jnp.int32)
    alpha_ref[...] = jax.nn.sigmoid(jnp.max(aff, axis=1))


def _routing(feats, cents):
    return pl.pallas_call(
        _routing_body,
        out_shape=(jax.ShapeDtypeStruct((N,), jnp.int32),
                   jax.ShapeDtypeStruct((N,), jnp.float32)),
    )(feats, cents)


def _sc_gather(table, idx3, k):
    d = table.shape[1]
    mesh = plsc.VectorSubcoreMesh(core_axis_name="c", subcore_axis_name="s")
    info = plsc.get_sparse_core_info()
    nc = info.num_cores

    @functools.partial(
        pl.kernel, mesh=mesh,
        out_type=jax.ShapeDtypeStruct((NW * k * CH, d), jnp.float32),
        scratch_types=[
            pltpu.VMEM((k, CH), jnp.int32),
            pltpu.VMEM((CH, d), jnp.float32),
            pltpu.VMEM((CH, d), jnp.float32),
            pltpu.SemaphoreType.DMA,
            pltpu.SemaphoreType.DMA,
        ],
    )
    def run(table_hbm, idx_hbm, out_hbm, idx_v, r0, r1, gsem, ssem):
        wid = lax.axis_index("s") * nc + lax.axis_index("c")
        base = wid * k * CH
        pltpu.sync_copy(idx_hbm.at[wid], idx_v)
        bufs = (r0, r1)
        g = [None] * k
        s = [None] * k
        g[0] = pltpu.async_copy(table_hbm.at[idx_v.at[0]], bufs[0], gsem)
        for c in range(k):
            if c + 1 < k:
                if c >= 1:
                    s[c - 1].wait()
                g[c + 1] = pltpu.async_copy(
                    table_hbm.at[idx_v.at[c + 1]], bufs[(c + 1) % 2], gsem)
            g[c].wait()
            s[c] = pltpu.async_copy(
                bufs[c % 2], out_hbm.at[pl.ds(base + c * CH, CH)], ssem)
        s[k - 1].wait()
        if k >= 2:
            s[k - 2].wait()

    return run(table, idx3)


def _ffn_body(te_ref, act_ref, x_ref, a_ref, nw_ref, nb_ref,
              w1_ref, b1_ref, w2_ref, b2_ref, out_ref):
    g = pl.program_id(0)

    @pl.when(g < act_ref[0])
    def _():
        x = x_ref[...]
        mu = jnp.mean(x, axis=1, keepdims=True)
        var = jnp.mean((x - mu) ** 2, axis=1, keepdims=True)
        xh = (x - mu) * lax.rsqrt(var + 1e-5) * nw_ref[0] + nb_ref[0]
        h = lax.dot_general(xh.astype(jnp.bfloat16), w1_ref[0],
                            (((1,), (1,)), ((), ())),
                            preferred_element_type=jnp.float32)
        h = jnp.maximum(h + b1_ref[0], 0.0)
        y = lax.dot_general(h.astype(jnp.bfloat16), w2_ref[0],
                            (((1,), (1,)), ((), ())),
                            preferred_element_type=jnp.float32)
        out_ref[...] = x + a_ref[...] * (y + b2_ref[0])


def _grouped_ffn(te, act, x_p, alpha_p, norm_w, norm_b, ff1_w, ff1_b, ff2_w, ff2_b):
    grid_spec = pltpu.PrefetchScalarGridSpec(
        num_scalar_prefetch=2,
        grid=(G,),
        in_specs=[
            pl.BlockSpec((T, D), lambda g, te, act: (g, 0)),
            pl.BlockSpec((T, 1), lambda g, te, act: (g, 0)),
            pl.BlockSpec((1, 1, D), lambda g, te, act: (te[g], 0, 0)),
            pl.BlockSpec((1, 1, D), lambda g, te, act: (te[g], 0, 0)),
            pl.BlockSpec((1, DFF, D), lambda g, te, act: (te[g], 0, 0)),
            pl.BlockSpec((1, 1, DFF), lambda g, te, act: (te[g], 0, 0)),
            pl.BlockSpec((1, D, DFF), lambda g, te, act: (te[g], 0, 0)),
            pl.BlockSpec((1, 1, D), lambda g, te, act: (te[g], 0, 0)),
        ],
        out_specs=pl.BlockSpec((T, D), lambda g, te, act: (g, 0)),
    )
    return pl.pallas_call(
        _ffn_body,
        grid_spec=grid_spec,
        out_shape=jax.ShapeDtypeStruct((G * T, D), jnp.float32),
        compiler_params=pltpu.CompilerParams(
            dimension_semantics=("arbitrary",)),
    )(te, act, x_p, alpha_p, norm_w, norm_b, ff1_w, ff1_b, ff2_w, ff2_b)


def kernel(input_features, expert_centroids, norm_w, norm_b,
           ff1_w, ff1_b, ff2_w, ff2_b):
    feats = input_features.reshape(N, D)

    eid, alpha = _routing(feats, expert_centroids)

    oh = (eid[:, None] == jnp.arange(E, dtype=jnp.int32)[None, :]).astype(jnp.int32)
    csum = jnp.cumsum(oh, axis=0)
    rank = jnp.sum((csum - oh) * oh, axis=1)
    counts = csum[-1]
    tiles_e = (counts + T - 1) // T
    tcum = jnp.cumsum(tiles_e)
    tile_start = tcum - tiles_e
    pos = tile_start[eid] * T + rank
    src_idx = (jnp.arange(G * T, dtype=jnp.int32) % N).at[pos].set(
        jnp.arange(N, dtype=jnp.int32))
    alpha_p = jnp.zeros(G * T, jnp.float32).at[pos].set(alpha)
    n_act = tcum[-1]
    gidx = jnp.arange(G, dtype=jnp.int32)
    te_raw = jnp.searchsorted(tcum, gidx, side="right").astype(jnp.int32)
    last_e = jnp.max(jnp.where(counts > 0, jnp.arange(E, dtype=jnp.int32), 0))
    te = jnp.where(gidx < n_act, jnp.minimum(te_raw, E - 1), last_e)
    act = n_act.reshape(1).astype(jnp.int32)

    x_p = _sc_gather(feats, src_idx.reshape(NW, (G * T) // (NW * CH), CH),
                     (G * T) // (NW * CH))
    y_p = _grouped_ffn(te, act, x_p, alpha_p.reshape(G * T, 1),
                       norm_w.reshape(E, 1, D), norm_b.reshape(E, 1, D),
                       ff1_w.astype(jnp.bfloat16), ff1_b.reshape(E, 1, DFF),
                       ff2_w.astype(jnp.bfloat16), ff2_b.reshape(E, 1, D))
    out = _sc_gather(y_p, pos.reshape(NW, N // (NW * CH), CH),
                     N // (NW * CH))
    return out.reshape(input_features.shape)
